# Initial kernel scaffold; baseline (speedup 1.0000x reference)
import jax
import jax.numpy as jnp
from jax.experimental import pallas as pl


def kernel(x, edge_index, W_l0, b_l0, W_r0, bn_g0, bn_b0, W_l1, b_l1, W_r1, bn_g1, bn_b1, W_l2, b_l2, W_r2, bn_g2, bn_b2, W_c0, b_c0, W_c1, b_c1, W_c2, b_c2):
    raise NotImplementedError("write your pallas kernel here")


# speedup vs baseline: 7.7831x; 7.7831x over previous
import functools

import jax
import jax.numpy as jnp
from jax import lax
from jax.experimental import pallas as pl
from jax.experimental.pallas import tpu as pltpu
from jax.experimental.pallas import tpu_sc as plsc

N = 50000
E = 800000
NC = 2
NS = 16
NW = NC * NS
EPT = E // NW
EB = 125
NB = EPT // EB
NPAD = 50048
RPT = NPAD // NS
SB = 40
NSB = NB // SB
BLK = 2000
GRID = N // BLK
EPS_BN = 1e-5


def _make_segsum(n_chunks, width):
    mesh = plsc.VectorSubcoreMesh(core_axis_name="c", subcore_axis_name="s")
    out_type = [jax.ShapeDtypeStruct((NC, NPAD, width), jnp.float32)
                for _ in range(n_chunks)]
    scratch = [
        pltpu.VMEM((EB, width), jnp.float32),
        pltpu.VMEM((EB, width), jnp.float32),
        pltpu.VMEM((SB, EB), jnp.int32),
        pltpu.VMEM((SB, EB), jnp.int32),
        pltpu.SemaphoreType.DMA,
        pltpu.SemaphoreType.DMA,
        pltpu.VMEM_SHARED((NPAD, width), jnp.float32),
    ]

    @functools.partial(
        pl.kernel, mesh=mesh, out_type=out_type, scratch_types=scratch,
        compiler_params=pltpu.CompilerParams(use_tc_tiling_on_sc=False))
    def seg_kernel(src_h, dst_h, zero_h, *rest):
        tables = rest[:n_chunks]
        outs = rest[n_chunks:2 * n_chunks]
        g0, g1, si, di, s0, s1, acc = rest[2 * n_chunks:]
        cid = lax.axis_index("c")
        sid = lax.axis_index("s")
        wid = cid * NS + sid
        r0 = sid * RPT
        rows = pl.ds(r0, RPT)

        for c in range(n_chunks):
            tab = tables[c]
            out = outs[c]
            pltpu.sync_copy(zero_h, acc.at[rows])
            plsc.subcore_barrier()

            def super_body(sb, carry):
                bat = pl.ds(sb * SB, SB)
                pltpu.sync_copy(src_h.at[wid, bat], si)
                pltpu.sync_copy(dst_h.at[wid, bat], di)
                pltpu.async_copy(tab.at[si.at[0]], g0, s0)

                def edge_body(i, c2):
                    b0 = 2 * i
                    b1 = b0 + 1
                    pltpu.async_copy(tab.at[si.at[b1]], g1, s1)
                    pltpu.make_async_copy(tab.at[si.at[b0]], g0, s0).wait()
                    pltpu.sync_copy(g0, acc.at[di.at[b0]], add=True)

                    @pl.when(b0 + 2 < SB)
                    def _():
                        pltpu.async_copy(tab.at[si.at[b0 + 2]], g0, s0)

                    pltpu.make_async_copy(tab.at[si.at[b1]], g1, s1).wait()
                    pltpu.sync_copy(g1, acc.at[di.at[b1]], add=True)
                    return c2

                lax.fori_loop(0, SB // 2, edge_body, 0)
                return carry

            lax.fori_loop(0, NSB, super_body, 0)
            plsc.subcore_barrier()
            pltpu.sync_copy(acc.at[rows], out.at[cid, rows])

    return seg_kernel


_SEGSUM_CACHE = {}


def _run_segsum(src3, dst3, tables, width):
    key = (len(tables), width)
    if key not in _SEGSUM_CACHE:
        _SEGSUM_CACHE[key] = _make_segsum(*key)
    zero_rows = jnp.zeros((RPT, width), jnp.float32)
    res = _SEGSUM_CACHE[key](src3, dst3, zero_rows, *tables)
    return res if isinstance(res, (list, tuple)) else (res,)


def _full(shape):
    return pl.BlockSpec(shape, lambda i: tuple(0 for _ in shape))


def _rows(width):
    return pl.BlockSpec((BLK, width), lambda i: (i, 0))


def _pair(width):
    return pl.BlockSpec((NC, BLK, width), lambda i: (0, i, 0))


def _norm_stats(step, raw, hn_ref, st_ref):
    nrm = jnp.sqrt(jnp.sum(raw * raw, axis=-1, keepdims=True))
    hn = raw / jnp.maximum(nrm, 1e-12)
    hn_ref[...] = hn

    @pl.when(step == 0)
    def _():
        st_ref[...] = jnp.zeros_like(st_ref)

    st_ref[0:1, :] += jnp.sum(hn, axis=0, keepdims=True)
    st_ref[1:2, :] += jnp.sum(hn * hn, axis=0, keepdims=True)


def _bn_apply(hn, st_ref, g_ref, b_ref):
    mu = st_ref[0:1, :] / N
    var = st_ref[1:2, :] / N - mu * mu
    scale = g_ref[...] / jnp.sqrt(var + EPS_BN)
    shift = b_ref[...] - mu * scale
    return hn * scale + shift


def _tc1_body(s0_ref, x_ref, wl_ref, bl_ref, wr_ref,
              hn_ref, st_ref, dinv_ref):
    i = pl.program_id(0)
    s = s0_ref[0] + s0_ref[1]
    dinv = 1.0 / jnp.maximum(s[:, 12:13], 1.0)
    mean = s[:, :12] * dinv
    raw = (jnp.dot(mean, wl_ref[...], preferred_element_type=jnp.float32)
           + bl_ref[...]
           + jnp.dot(x_ref[...], wr_ref[...],
                     preferred_element_type=jnp.float32))
    _norm_stats(i, raw, hn_ref, st_ref)
    dinv_ref[...] = dinv


def _tc2_body(hn_ref, st_ref, g_ref, b_ref, t0, t1, t2, t3):
    hp = jnp.maximum(_bn_apply(hn_ref[...], st_ref, g_ref, b_ref), 0.0)
    t0[...] = hp[:, 0:32]
    t1[...] = hp[:, 32:64]
    t2[...] = hp[:, 64:96]
    t3[...] = hp[:, 96:128]


def _tc3_body(s0, s1, s2, s3, dinv_ref, t0, t1, t2, t3,
              wl_ref, bl_ref, wr_ref, hn_ref, st_ref):
    i = pl.program_id(0)
    s = jnp.concatenate([s0[0] + s0[1], s1[0] + s1[1],
                         s2[0] + s2[1], s3[0] + s3[1]], axis=1)
    mean = s * dinv_ref[...]
    h0 = jnp.concatenate([t0[...], t1[...], t2[...], t3[...]], axis=1)
    raw = (jnp.dot(mean, wl_ref[...], preferred_element_type=jnp.float32)
           + bl_ref[...]
           + jnp.dot(h0, wr_ref[...], preferred_element_type=jnp.float32))
    _norm_stats(i, raw, hn_ref, st_ref)


def _tc4_body(hn_ref, st_ref, g_ref, b_ref, wl2_ref, bl2_ref, wr2_ref,
              v0, v1, r2_ref):
    hp = jnp.maximum(_bn_apply(hn_ref[...], st_ref, g_ref, b_ref), 0.0)
    v2 = jnp.dot(hp, wl2_ref[...], preferred_element_type=jnp.float32)
    v0[...] = v2[:, 0:32]
    v1[...] = v2[:, 32:64]
    r2_ref[...] = (jnp.dot(hp, wr2_ref[...],
                           preferred_element_type=jnp.float32)
                   + bl2_ref[...])


def _tc5_body(s0, s1, dinv_ref, r2_ref, hn_ref, st_ref):
    i = pl.program_id(0)
    s = jnp.concatenate([s0[0] + s0[1], s1[0] + s1[1]], axis=1)
    raw = s * dinv_ref[...] + r2_ref[...]
    _norm_stats(i, raw, hn_ref, st_ref)


def _tc6_body(hn_ref, st_ref, g_ref, b_ref, w0_ref, b0_ref,
              w1_ref, b1_ref, w2_ref, b2_ref, out_ref):
    hp = _bn_apply(hn_ref[...], st_ref, g_ref, b_ref)
    a = jnp.maximum(jnp.dot(hp, w0_ref[...],
                            preferred_element_type=jnp.float32)
                    + b0_ref[...], 0.0)
    a = jnp.maximum(jnp.dot(a, w1_ref[...],
                            preferred_element_type=jnp.float32)
                    + b1_ref[...], 0.0)
    out_ref[...] = (jnp.dot(a, w2_ref[...],
                            preferred_element_type=jnp.float32)
                    + b2_ref[...])


def _tc_call(body, in_specs, out_specs, out_shapes, args):
    return pl.pallas_call(
        body,
        grid=(GRID,),
        in_specs=in_specs,
        out_specs=out_specs,
        out_shape=out_shapes,
    )(*args)


def kernel(x, edge_index, W_l0, b_l0, W_r0, bn_g0, bn_b0,
           W_l1, b_l1, W_r1, bn_g1, bn_b1,
           W_l2, b_l2, W_r2, bn_g2, bn_b2,
           W_c0, b_c0, W_c1, b_c1, W_c2, b_c2):
    f32 = jnp.float32
    src3 = edge_index[0].reshape(NW, NB, EB)
    dst3 = edge_index[1].reshape(NW, NB, EB)
    tx = jnp.concatenate(
        [x, jnp.ones((N, 1), f32), jnp.zeros((N, 3), f32)], axis=1)

    (s0,) = _run_segsum(src3, dst3, (tx,), 16)

    hn0, st0, dinv = _tc_call(
        _tc1_body,
        [_pair(16), _rows(12), _full((12, 128)), _full((1, 128)),
         _full((12, 128))],
        [_rows(128), _full((8, 128)), _rows(1)],
        [jax.ShapeDtypeStruct((N, 128), f32),
         jax.ShapeDtypeStruct((8, 128), f32),
         jax.ShapeDtypeStruct((N, 1), f32)],
        (s0, x, W_l0.T, b_l0.reshape(1, 128), W_r0.T))

    t0 = _tc_call(
        _tc2_body,
        [_rows(128), _full((8, 128)), _full((1, 128)), _full((1, 128))],
        [_rows(32)] * 4,
        [jax.ShapeDtypeStruct((N, 32), f32)] * 4,
        (hn0, st0, bn_g0.reshape(1, 128), bn_b0.reshape(1, 128)))

    s1 = _run_segsum(src3, dst3, t0, 32)

    hn1, st1 = _tc_call(
        _tc3_body,
        [_pair(32)] * 4 + [_rows(1)] + [_rows(32)] * 4
        + [_full((128, 128)), _full((1, 128)), _full((128, 128))],
        [_rows(128), _full((8, 128))],
        [jax.ShapeDtypeStruct((N, 128), f32),
         jax.ShapeDtypeStruct((8, 128), f32)],
        (*s1, dinv, *t0, W_l1.T, b_l1.reshape(1, 128), W_r1.T))

    v2c0, v2c1, r2 = _tc_call(
        _tc4_body,
        [_rows(128), _full((8, 128)), _full((1, 128)), _full((1, 128)),
         _full((128, 64)), _full((1, 64)), _full((128, 64))],
        [_rows(32), _rows(32), _rows(64)],
        [jax.ShapeDtypeStruct((N, 32), f32),
         jax.ShapeDtypeStruct((N, 32), f32),
         jax.ShapeDtypeStruct((N, 64), f32)],
        (hn1, st1, bn_g1.reshape(1, 128), bn_b1.reshape(1, 128),
         W_l2.T, b_l2.reshape(1, 64), W_r2.T))

    s2 = _run_segsum(src3, dst3, (v2c0, v2c1), 32)

    hn2, st2 = _tc_call(
        _tc5_body,
        [_pair(32)] * 2 + [_rows(1), _rows(64)],
        [_rows(64), _full((8, 64))],
        [jax.ShapeDtypeStruct((N, 64), f32),
         jax.ShapeDtypeStruct((8, 64), f32)],
        (*s2, dinv, r2))

    (logits,) = _tc_call(
        _tc6_body,
        [_rows(64), _full((8, 64)), _full((1, 64)), _full((1, 64)),
         _full((64, 64)), _full((1, 64)), _full((64, 32)), _full((1, 32)),
         _full((32, 1)), _full((1, 1))],
        [_rows(1)],
        [jax.ShapeDtypeStruct((N, 1), f32)],
        (hn2, st2, bn_g2.reshape(1, 64), bn_b2.reshape(1, 64),
         W_c0.T, b_c0.reshape(1, 64), W_c1.T, b_c1.reshape(1, 32),
         W_c2.T, b_c2.reshape(1, 1)))

    return logits[:, 0]
